# baseline (speedup 1.0000x reference)
import functools

import jax
import jax.numpy as jnp
from jax import lax
from jax.experimental import pallas as pl
from jax.experimental.pallas import tpu as pltpu
from jax.experimental.pallas import tpu_sc as plsc

N = 10000
NPAD = 10240
RB = 128
NBLK = NPAD // RB
K_NN = 8
STD_THRES = 2.0
RADIUS = 0.01
H, W = 256, 384
PIX = H * W
WORDS = 4 * PIX
NS = 16
PTS_PER_TILE = NPAD // NS
WCHUNKS = PTS_PER_TILE * 4 // 128
WROWS = 24
PIXW_PER_TILE = WORDS // NS
PPT = PIX // NS
NVEC = PPT // 16
PAD_COORD = 1e4


def _knn_body(pts_ref, ptsT_ref, sqc_ref, sqr_ref, out_ref):
    btb = ptsT_ref[...].astype(jnp.bfloat16)
    sqr = sqr_ref[...]

    inf8 = jnp.full((8, 128), jnp.inf, jnp.float32)
    lists = (inf8,) * (2 * (K_NN + 1))

    def insert(i, ls):
        la = list(ls[:K_NN + 1])
        lb = list(ls[K_NN + 1:])
        base = i * 64
        d = lax.dot_general(pts_ref[pl.ds(base, 64), :].astype(jnp.bfloat16),
                            btb, (((1,), (0,)), ((), ())),
                            preferred_element_type=jnp.float32)
        d2c = (sqc_ref[pl.ds(base, 64), :] + sqr) - 2.0 * d
        for j in range(0, 8, 2):
            va = d2c[j * 8:(j + 1) * 8, :]
            vb = d2c[(j + 1) * 8:(j + 2) * 8, :]
            la[K_NN] = jnp.minimum(la[K_NN], va)
            lb[K_NN] = jnp.minimum(lb[K_NN], vb)
            for t in range(K_NN, 0, -1):
                loa = jnp.minimum(la[t - 1], la[t])
                hia = jnp.maximum(la[t - 1], la[t])
                lob = jnp.minimum(lb[t - 1], lb[t])
                hib = jnp.maximum(lb[t - 1], lb[t])
                la[t - 1], la[t] = loa, hia
                lb[t - 1], lb[t] = lob, hib
        return tuple(la) + tuple(lb)

    lists = lax.fori_loop(0, NPAD // 64, insert, lists)
    small = jnp.concatenate(lists, axis=0)

    taken = jnp.zeros((1, 128), jnp.float32)
    s9 = jnp.zeros((1, 128), jnp.float32)
    m0 = None
    for p in range(K_NN + 1):
        m = jnp.min(small, axis=0, keepdims=True)
        if p == 0:
            m0 = m
        eq = small == m
        c = jnp.sum(eq.astype(jnp.float32), axis=0, keepdims=True)
        rem = (K_NN + 1.0) - taken
        take = jnp.minimum(c, rem)
        contrib = jnp.where(rem > 0.0, take * jnp.maximum(m, 0.0), 0.0)
        s9 = s9 + contrib
        taken = taken + take
        if p < K_NN:
            small = jnp.where(eq, jnp.float32(jnp.inf), small)
    out_ref[...] = ((s9 - jnp.maximum(m0, 0.0)) * (1.0 / K_NN)).reshape(
        1, 1, 128)


def _knn_call(pts, ptsT, sqc, sqr):
    return pl.pallas_call(
        _knn_body,
        grid=(NPAD // 128,),
        in_specs=[
            pl.BlockSpec((NPAD, 8), lambda i: (0, 0)),
            pl.BlockSpec((8, 128), lambda i: (0, i)),
            pl.BlockSpec((NPAD, 1), lambda i: (0, 0)),
            pl.BlockSpec((1, 128), lambda i: (0, i)),
        ],
        out_specs=pl.BlockSpec((1, 1, 128), lambda i: (i, 0, 0)),
        out_shape=jax.ShapeDtypeStruct((NPAD // 128, 1, 128), jnp.float32),
    )(pts, ptsT, sqc, sqr)


def _proj_body(avg_ref, p6_ref, cam_ref, vals_ref, idx_ref):
    avg = avg_ref[...]
    rows = lax.broadcasted_iota(jnp.int32, (80, 128), 0)
    cols = lax.broadcasted_iota(jnp.int32, (80, 128), 1)
    lin = rows * 128 + cols
    valid_pt = lin < N
    bits = lax.bitcast_convert_type(avg, jnp.int32)

    def select(kth):
        ans = jnp.int32(0)
        for b in range(30, -1, -1):
            t = ans + jnp.int32(1 << b)
            cnt = jnp.sum(jnp.where((bits < t) & valid_pt, 1, 0))
            ans = jnp.where(cnt < kth, t, ans)
        return ans

    lo = lax.bitcast_convert_type(select(N // 2), jnp.float32)
    hi = lax.bitcast_convert_type(select(N // 2 + 1), jnp.float32)
    med = (lo + hi) * 0.5
    vf = valid_pt.astype(jnp.float32)
    mean = jnp.sum(avg * vf) * (1.0 / N)
    var = jnp.sum(jnp.where(valid_pt, (avg - mean) * (avg - mean), 0.0)) * (1.0 / (N - 1))
    thres = med + jnp.sqrt(var) * STD_THRES
    pw = jnp.where((avg < thres) & valid_pt, 1.0, 0.0)

    camx = p6_ref[0]
    camy = p6_ref[1]
    camz = p6_ref[2]
    pr = p6_ref[3]
    pg = p6_ref[4]
    pb = p6_ref[5]
    c = cam_ref[...]
    k00, k02, k11, k12 = c[0, 0], c[0, 1], c[0, 2], c[0, 3]

    zs = jnp.where(jnp.abs(camz) < 1e-8, 1e-8, camz)
    u = k00 * camx / zs + k02
    v = k11 * camy / zs + k12
    valid = (camz > 1e-6) & (u >= 0.0) & (u < float(W)) & (v >= 0.0) & (v < float(H))
    ui = jnp.clip(jnp.floor(u + 0.5), 0.0, float(W - 1))
    vi = jnp.clip(jnp.floor(v + 0.5), 0.0, float(H - 1))
    rad_px = max(RADIUS * 0.5 * max(H, W), 1.5)
    du = u - ui
    dv = v - vi
    wgt = jnp.maximum(1.0 - (du * du + dv * dv) / (rad_px * rad_px), 0.0)
    wgt = wgt * valid.astype(jnp.float32) * pw
    vals_ref[0] = wgt * pr
    vals_ref[1] = wgt * pg
    vals_ref[2] = wgt * pb
    vals_ref[3] = wgt
    idx_ref[...] = (vi * float(W) + ui).astype(jnp.int32)


def _proj_call(avg2, p6, cam):
    return pl.pallas_call(
        _proj_body,
        out_shape=[
            jax.ShapeDtypeStruct((4, 80, 128), jnp.float32),
            jax.ShapeDtypeStruct((80, 128), jnp.int32),
        ],
    )(avg2, p6, cam)


def _sc_body(idxw_hbm, valsw_hbm, zeros_hbm, out_hbm,
             idx_v, vals_v, work_v, acc):
    sid = lax.axis_index("s")
    wbase = sid * PIXW_PER_TILE
    pltpu.sync_copy(zeros_hbm.at[pl.ds(wbase, PIXW_PER_TILE)],
                    acc.at[pl.ds(wbase, PIXW_PER_TILE)])
    pltpu.sync_copy(idxw_hbm.at[pl.ds(sid * WROWS, WROWS)], idx_v)
    pltpu.sync_copy(valsw_hbm.at[pl.ds(sid * WROWS, WROWS)], vals_v)
    plsc.subcore_barrier()
    for j in range(WCHUNKS):
        pltpu.sync_copy(vals_v.at[j], acc.at[idx_v.at[j]], add=True)
    plsc.subcore_barrier()
    pbase = sid * PPT
    for ch in range(4):
        pltpu.sync_copy(acc.at[pl.ds(ch * PIX + pbase, PPT)],
                        work_v.at[pl.ds(ch * PPT, PPT)])

    def body(i, carry):
        o = i * 16
        den = work_v[pl.ds(3 * PPT + o, 16)]
        dsafe = jnp.maximum(den, 1e-8)
        work_v[pl.ds(0 * PPT + o, 16)] = work_v[pl.ds(0 * PPT + o, 16)] / dsafe
        work_v[pl.ds(1 * PPT + o, 16)] = work_v[pl.ds(1 * PPT + o, 16)] / dsafe
        work_v[pl.ds(2 * PPT + o, 16)] = work_v[pl.ds(2 * PPT + o, 16)] / dsafe
        work_v[pl.ds(3 * PPT + o, 16)] = jnp.where(
            den > 0.0, jnp.float32(1.0), jnp.float32(0.0))
        return carry

    lax.fori_loop(0, NVEC, body, 0)
    for ch in range(4):
        pltpu.sync_copy(work_v.at[pl.ds(ch * PPT, PPT)],
                        out_hbm.at[pl.ds(ch * PIX + pbase, PPT)])


@functools.cache
def _sc_render():
    mesh = plsc.VectorSubcoreMesh(
        core_axis_name="c", subcore_axis_name="s", num_cores=1,
        num_subcores=NS)
    return pl.kernel(
        _sc_body,
        out_type=jax.ShapeDtypeStruct((WORDS,), jnp.float32),
        mesh=mesh,
        scratch_types=[
            pltpu.VMEM((WROWS, 128), jnp.int32),
            pltpu.VMEM((WROWS, 128), jnp.float32),
            pltpu.VMEM((PIXW_PER_TILE,), jnp.float32),
            pltpu.VMEM_SHARED((WORDS,), jnp.float32),
        ],
    )


def kernel(tgt_h, tgt_w, flat_tgt_cam, st_pcl_rgb):
    f32 = jnp.float32
    pcl = st_pcl_rgb[:, :3]
    rgb = st_pcl_rgb[:, 3:]
    Kmat = flat_tgt_cam[2:18].reshape(4, 4)
    c2w = flat_tgt_cam[18:34].reshape(4, 4)
    w2c = jnp.linalg.inv(c2w)

    p3 = jnp.concatenate([pcl, jnp.full((NPAD - N, 3), PAD_COORD, f32)], axis=0)
    pts = jnp.pad(p3, ((0, 0), (0, 5)))
    sqv = jnp.sum(p3 * p3, axis=1)
    avg = _knn_call(pts, pts.T, sqv.reshape(NPAD, 1),
                    sqv.reshape(1, NPAD))

    rgbp = jnp.concatenate([rgb, jnp.zeros((NPAD - N, 3), f32)], axis=0)
    camf = pcl @ w2c[:3, :3].T + w2c[:3, 3]
    camp = jnp.concatenate(
        [camf, jnp.full((NPAD - N, 3), PAD_COORD, f32)], axis=0)
    p6 = jnp.concatenate([camp, rgbp], axis=1).T.reshape(6, 80, 128)
    cam_vec = jnp.stack(
        [Kmat[0, 0], Kmat[0, 2], Kmat[1, 1], Kmat[1, 2]])
    cam = jnp.pad(cam_vec.astype(f32), (0, 124)).reshape(1, 128)
    vals4, idx = _proj_call(avg.reshape(80, 128), p6, cam)

    fidx = idx.reshape(NPAD)
    valsw = vals4.reshape(4, NPAD).T.reshape(NS, WCHUNKS, 128)
    valsw = jnp.pad(valsw, ((0, 0), (0, WROWS - WCHUNKS), (0, 0))).reshape(
        NS * WROWS, 128)
    idxw = (fidx[:, None]
            + (jnp.arange(4, dtype=jnp.int32) * PIX)[None, :]).reshape(
                NS, WCHUNKS, 128)
    idxw = jnp.pad(idxw, ((0, 0), (0, WROWS - WCHUNKS), (0, 0))).reshape(
        NS * WROWS, 128)
    zeros = jnp.zeros((WORDS,), f32)
    out = _sc_render()(idxw, valsw, zeros)
    arr = out.reshape(4, H, W)
    mesh_img = jnp.transpose(arr[:3], (1, 2, 0))
    mesh_mask = arr[3][:, :, None]
    return (mesh_img, mesh_mask)


# speedup vs baseline: 3.5252x; 3.5252x over previous
import functools

import jax
import jax.numpy as jnp
from jax import lax
from jax.experimental import pallas as pl
from jax.experimental.pallas import tpu as pltpu
from jax.experimental.pallas import tpu_sc as plsc

N = 10000
NPAD = 10240
RB = 128
NBLK = NPAD // RB
K_NN = 8
STD_THRES = 2.0
RADIUS = 0.01
H, W = 256, 384
PIX = H * W
WORDS = 4 * PIX
NS = 16
PTS_PER_TILE = NPAD // NS
WCHUNKS = PTS_PER_TILE * 4 // 128
WROWS = 24
PIXW_PER_TILE = WORDS // NS
PPT = PIX // NS
NVEC = PPT // 16
PAD_COORD = 1e4


def _knn_body(pts_ref, ptsT_ref, sqc_ref, sqr_ref, out_ref, work_ref):
    bt = ptsT_ref[...]
    dot = lax.dot_general(pts_ref[...].astype(jnp.bfloat16),
                          bt.astype(jnp.bfloat16),
                          (((1,), (0,)), ((), ())),
                          preferred_element_type=jnp.float32)
    work_ref[...] = (sqc_ref[...] + sqr_ref[...]) - 2.0 * dot

    inf8 = jnp.full((8, 128), jnp.inf, jnp.float32)
    lists = (inf8,) * (2 * (K_NN + 1))

    def insert(i, ls):
        la = list(ls[:K_NN + 1])
        lb = list(ls[K_NN + 1:])
        base = i * 64
        for j in range(0, 8, 2):
            va = work_ref[pl.ds(base + j * 8, 8), :]
            vb = work_ref[pl.ds(base + (j + 1) * 8, 8), :]
            la[K_NN] = jnp.minimum(la[K_NN], va)
            lb[K_NN] = jnp.minimum(lb[K_NN], vb)
            for t in range(K_NN, 0, -1):
                loa = jnp.minimum(la[t - 1], la[t])
                hia = jnp.maximum(la[t - 1], la[t])
                lob = jnp.minimum(lb[t - 1], lb[t])
                hib = jnp.maximum(lb[t - 1], lb[t])
                la[t - 1], la[t] = loa, hia
                lb[t - 1], lb[t] = lob, hib
        return tuple(la) + tuple(lb)

    lists = lax.fori_loop(0, NPAD // 64, insert, lists)
    small = jnp.concatenate(lists, axis=0)

    taken = jnp.zeros((1, 128), jnp.float32)
    s9 = jnp.zeros((1, 128), jnp.float32)
    m0 = None
    for p in range(K_NN + 1):
        m = jnp.min(small, axis=0, keepdims=True)
        if p == 0:
            m0 = m
        eq = small == m
        c = jnp.sum(eq.astype(jnp.float32), axis=0, keepdims=True)
        rem = (K_NN + 1.0) - taken
        take = jnp.minimum(c, rem)
        contrib = jnp.where(rem > 0.0, take * jnp.maximum(m, 0.0), 0.0)
        s9 = s9 + contrib
        taken = taken + take
        if p < K_NN:
            small = jnp.where(eq, jnp.float32(jnp.inf), small)
    out_ref[...] = ((s9 - jnp.maximum(m0, 0.0)) * (1.0 / K_NN)).reshape(
        1, 1, 128)


def _knn_call(pts, ptsT, sqc, sqr):
    return pl.pallas_call(
        _knn_body,
        grid=(NPAD // 128,),
        in_specs=[
            pl.BlockSpec((NPAD, 8), lambda i: (0, 0)),
            pl.BlockSpec((8, 128), lambda i: (0, i)),
            pl.BlockSpec((NPAD, 1), lambda i: (0, 0)),
            pl.BlockSpec((1, 128), lambda i: (0, i)),
        ],
        out_specs=pl.BlockSpec((1, 1, 128), lambda i: (i, 0, 0)),
        out_shape=jax.ShapeDtypeStruct((NPAD // 128, 1, 128), jnp.float32),
        scratch_shapes=[pltpu.VMEM((NPAD, 128), jnp.float32)],
    )(pts, ptsT, sqc, sqr)


def _proj_body(avg_ref, p6_ref, cam_ref, vals_ref, idx_ref):
    avg = avg_ref[...]
    rows = lax.broadcasted_iota(jnp.int32, (80, 128), 0)
    cols = lax.broadcasted_iota(jnp.int32, (80, 128), 1)
    lin = rows * 128 + cols
    valid_pt = lin < N
    bits = lax.bitcast_convert_type(avg, jnp.int32)

    def select(kth):
        ans = jnp.int32(0)
        for b in range(30, -1, -1):
            t = ans + jnp.int32(1 << b)
            cnt = jnp.sum(jnp.where((bits < t) & valid_pt, 1, 0))
            ans = jnp.where(cnt < kth, t, ans)
        return ans

    lo = lax.bitcast_convert_type(select(N // 2), jnp.float32)
    hi = lax.bitcast_convert_type(select(N // 2 + 1), jnp.float32)
    med = (lo + hi) * 0.5
    vf = valid_pt.astype(jnp.float32)
    mean = jnp.sum(avg * vf) * (1.0 / N)
    var = jnp.sum(jnp.where(valid_pt, (avg - mean) * (avg - mean), 0.0)) * (1.0 / (N - 1))
    thres = med + jnp.sqrt(var) * STD_THRES
    pw = jnp.where((avg < thres) & valid_pt, 1.0, 0.0)

    camx = p6_ref[0]
    camy = p6_ref[1]
    camz = p6_ref[2]
    pr = p6_ref[3]
    pg = p6_ref[4]
    pb = p6_ref[5]
    c = cam_ref[...]
    k00, k02, k11, k12 = c[0, 0], c[0, 1], c[0, 2], c[0, 3]

    zs = jnp.where(jnp.abs(camz) < 1e-8, 1e-8, camz)
    u = k00 * camx / zs + k02
    v = k11 * camy / zs + k12
    valid = (camz > 1e-6) & (u >= 0.0) & (u < float(W)) & (v >= 0.0) & (v < float(H))
    ui = jnp.clip(jnp.floor(u + 0.5), 0.0, float(W - 1))
    vi = jnp.clip(jnp.floor(v + 0.5), 0.0, float(H - 1))
    rad_px = max(RADIUS * 0.5 * max(H, W), 1.5)
    du = u - ui
    dv = v - vi
    wgt = jnp.maximum(1.0 - (du * du + dv * dv) / (rad_px * rad_px), 0.0)
    wgt = wgt * valid.astype(jnp.float32) * pw
    vals_ref[0] = wgt * pr
    vals_ref[1] = wgt * pg
    vals_ref[2] = wgt * pb
    vals_ref[3] = wgt
    idx_ref[...] = (vi * float(W) + ui).astype(jnp.int32)


def _proj_call(avg2, p6, cam):
    return pl.pallas_call(
        _proj_body,
        out_shape=[
            jax.ShapeDtypeStruct((4, 80, 128), jnp.float32),
            jax.ShapeDtypeStruct((80, 128), jnp.int32),
        ],
    )(avg2, p6, cam)


def _sc_body(idxw_hbm, valsw_hbm, zeros_hbm, out_hbm,
             idx_v, vals_v, work_v, acc):
    sid = lax.axis_index("s")
    wbase = sid * PIXW_PER_TILE
    pltpu.sync_copy(zeros_hbm.at[pl.ds(wbase, PIXW_PER_TILE)],
                    acc.at[pl.ds(wbase, PIXW_PER_TILE)])
    pltpu.sync_copy(idxw_hbm.at[pl.ds(sid * WROWS, WROWS)], idx_v)
    pltpu.sync_copy(valsw_hbm.at[pl.ds(sid * WROWS, WROWS)], vals_v)
    plsc.subcore_barrier()
    for j in range(WCHUNKS):
        pltpu.sync_copy(vals_v.at[j], acc.at[idx_v.at[j]], add=True)
    plsc.subcore_barrier()
    pbase = sid * PPT
    for ch in range(4):
        pltpu.sync_copy(acc.at[pl.ds(ch * PIX + pbase, PPT)],
                        work_v.at[pl.ds(ch * PPT, PPT)])

    def body(i, carry):
        o = i * 16
        den = work_v[pl.ds(3 * PPT + o, 16)]
        dsafe = jnp.maximum(den, 1e-8)
        work_v[pl.ds(0 * PPT + o, 16)] = work_v[pl.ds(0 * PPT + o, 16)] / dsafe
        work_v[pl.ds(1 * PPT + o, 16)] = work_v[pl.ds(1 * PPT + o, 16)] / dsafe
        work_v[pl.ds(2 * PPT + o, 16)] = work_v[pl.ds(2 * PPT + o, 16)] / dsafe
        work_v[pl.ds(3 * PPT + o, 16)] = jnp.where(
            den > 0.0, jnp.float32(1.0), jnp.float32(0.0))
        return carry

    lax.fori_loop(0, NVEC, body, 0)
    for ch in range(4):
        pltpu.sync_copy(work_v.at[pl.ds(ch * PPT, PPT)],
                        out_hbm.at[pl.ds(ch * PIX + pbase, PPT)])


@functools.cache
def _sc_render():
    mesh = plsc.VectorSubcoreMesh(
        core_axis_name="c", subcore_axis_name="s", num_cores=1,
        num_subcores=NS)
    return pl.kernel(
        _sc_body,
        out_type=jax.ShapeDtypeStruct((WORDS,), jnp.float32),
        mesh=mesh,
        scratch_types=[
            pltpu.VMEM((WROWS, 128), jnp.int32),
            pltpu.VMEM((WROWS, 128), jnp.float32),
            pltpu.VMEM((PIXW_PER_TILE,), jnp.float32),
            pltpu.VMEM_SHARED((WORDS,), jnp.float32),
        ],
    )


def kernel(tgt_h, tgt_w, flat_tgt_cam, st_pcl_rgb):
    f32 = jnp.float32
    pcl = st_pcl_rgb[:, :3]
    rgb = st_pcl_rgb[:, 3:]
    Kmat = flat_tgt_cam[2:18].reshape(4, 4)
    c2w = flat_tgt_cam[18:34].reshape(4, 4)
    w2c = jnp.linalg.inv(c2w)

    p3 = jnp.concatenate([pcl, jnp.full((NPAD - N, 3), PAD_COORD, f32)], axis=0)
    pts = jnp.pad(p3, ((0, 0), (0, 5)))
    sqv = jnp.sum(p3 * p3, axis=1)
    avg = _knn_call(pts, pts.T, sqv.reshape(NPAD, 1),
                    sqv.reshape(1, NPAD))

    rgbp = jnp.concatenate([rgb, jnp.zeros((NPAD - N, 3), f32)], axis=0)
    camf = pcl @ w2c[:3, :3].T + w2c[:3, 3]
    camp = jnp.concatenate(
        [camf, jnp.full((NPAD - N, 3), PAD_COORD, f32)], axis=0)
    p6 = jnp.concatenate([camp, rgbp], axis=1).T.reshape(6, 80, 128)
    cam_vec = jnp.stack(
        [Kmat[0, 0], Kmat[0, 2], Kmat[1, 1], Kmat[1, 2]])
    cam = jnp.pad(cam_vec.astype(f32), (0, 124)).reshape(1, 128)
    vals4, idx = _proj_call(avg.reshape(80, 128), p6, cam)

    fidx = idx.reshape(NPAD)
    valsw = vals4.reshape(4, NPAD).T.reshape(NS, WCHUNKS, 128)
    valsw = jnp.pad(valsw, ((0, 0), (0, WROWS - WCHUNKS), (0, 0))).reshape(
        NS * WROWS, 128)
    idxw = (fidx[:, None]
            + (jnp.arange(4, dtype=jnp.int32) * PIX)[None, :]).reshape(
                NS, WCHUNKS, 128)
    idxw = jnp.pad(idxw, ((0, 0), (0, WROWS - WCHUNKS), (0, 0))).reshape(
        NS * WROWS, 128)
    zeros = jnp.zeros((WORDS,), f32)
    out = _sc_render()(idxw, valsw, zeros)
    arr = out.reshape(4, H, W)
    mesh_img = jnp.transpose(arr[:3], (1, 2, 0))
    mesh_mask = arr[3][:, :, None]
    return (mesh_img, mesh_mask)
